# Initial kernel scaffold; baseline (speedup 1.0000x reference)
import jax
import jax.numpy as jnp
from jax.experimental import pallas as pl


def kernel(input_ids, factor1_table, factor2_table, frequency_basis, hash_matrix, W, b):
    raise NotImplementedError("write your pallas kernel here")


# speedup vs baseline: 7.3452x; 7.3452x over previous
import functools

import jax
import jax.numpy as jnp
import numpy as np
from jax import lax
from jax.experimental import pallas as pl
from jax.experimental.pallas import tpu as pltpu
from jax.experimental.pallas import tpu_sc as plsc

_VOCAB = 1000000
_EMBED = 64
_NFREQ = 256
_F1 = 1001
_F2 = 1000
_NT = 16384 * 200
_NW = 32
_TPW = _NT // _NW
_C = 512
_NCHUNK = _TPW // _C
_KSUB = _C // 128


def _tables_body(f1_ref, f2_ref, fb_ref, w_ref, b_ref, t1_ref, t2_ref, tf_ref):
    w = w_ref[...]
    dn = (((1,), (1,)), ((), ()))
    hi = jax.lax.Precision.HIGHEST
    t1_ref[...] = lax.dot_general(f1_ref[...], w[:, :32], dn, precision=hi)
    t2_ref[...] = lax.dot_general(f2_ref[...], w[:, 32:], dn, precision=hi)
    tf_ref[...] = 0.1 * lax.dot_general(fb_ref[...], w, dn, precision=hi) + b_ref[...]


@jax.jit
def _make_tables(f1, f2, fb, w, b):
    return pl.pallas_call(
        _tables_body,
        out_shape=(
            jax.ShapeDtypeStruct((_F1, _EMBED), jnp.float32),
            jax.ShapeDtypeStruct((_F2, _EMBED), jnp.float32),
            jax.ShapeDtypeStruct((_NFREQ, _EMBED), jnp.float32),
        ),
    )(f1, f2, fb, w, b.reshape(1, _EMBED))


def _pack_bf16(t):
    bits = lax.bitcast_convert_type(t.astype(jnp.bfloat16), jnp.uint16)
    packed = bits[:, 0::2].astype(jnp.uint32) | (bits[:, 1::2].astype(jnp.uint32) << 16)
    return lax.bitcast_convert_type(packed, jnp.int32).reshape(-1)


@functools.cache
def _build_sc_lookup():
    mesh = plsc.VectorSubcoreMesh(
        core_axis_name="c", subcore_axis_name="s", num_cores=2, num_subcores=16
    )

    @functools.partial(
        pl.kernel,
        out_type=jax.ShapeDtypeStruct((_NT * _EMBED,), jnp.float32),
        mesh=mesh,
        scratch_types=[
            pltpu.VMEM((_F1 * 32,), jnp.int32),
            pltpu.VMEM((_F2 * 32,), jnp.int32),
            pltpu.VMEM((_NFREQ * 32,), jnp.int32),
            pltpu.VMEM((_C,), jnp.int32),
            pltpu.VMEM((_C,), jnp.int32),
            pltpu.VMEM((_C * _EMBED,), jnp.float32),
            pltpu.SemaphoreType.DMA,
        ],
        compiler_params=pltpu.CompilerParams(needs_layout_passes=False),
    )
    def sc_lookup(ids_hbm, t1_hbm, t2_hbm, tf_hbm, hash_hbm, out_hbm,
                  t1_v, t2_v, tf_v, ids_v, h_v, out_v, sem):
        cid = lax.axis_index("c")
        sid = lax.axis_index("s")
        wid = sid * 2 + cid
        pltpu.sync_copy(t1_hbm, t1_v)
        pltpu.sync_copy(t2_hbm, t2_v)
        pltpu.sync_copy(tf_hbm, tf_v)
        tok0 = wid * _TPW

        @pl.loop(0, _NCHUNK)
        def _chunk(ci):
            t0 = tok0 + ci * _C
            pltpu.sync_copy(ids_hbm.at[pl.ds(t0, _C)], ids_v)
            cps = [
                pltpu.async_copy(
                    hash_hbm.at[ids_v.at[pl.ds(j * 128, 128)]],
                    h_v.at[pl.ds(j * 128, 128)],
                    sem,
                )
                for j in range(_KSUB)
            ]
            for cp in cps:
                cp.wait()

            @pl.loop(0, _C // 16)
            def _grp(g):
                v = ids_v[pl.ds(g * 16, 16)]
                h = h_v[pl.ds(g * 16, 16)]
                vf = v.astype(jnp.float32)
                q0 = (vf * np.float32(1.0 / _F1)).astype(jnp.int32)
                r = v - q0 * _F1
                one = jnp.int32(1)
                zero = jnp.int32(0)
                q = q0 + jnp.where(r >= _F1, one, zero) - jnp.where(r < 0, one, zero)
                i1 = v - q * _F1
                i2 = jnp.minimum(q, _F2 - 1)
                b1 = i1 << 5
                b2 = i2 << 5
                b3 = h << 5
                obase = g * (16 * _EMBED) + (lax.iota(jnp.int32, 16) << 6)
                for p in range(32):
                    w1 = plsc.load_gather(t1_v, [b1 + p])
                    w2 = plsc.load_gather(t2_v, [b2 + p])
                    w3 = plsc.load_gather(tf_v, [b3 + p])
                    sb = (
                        plsc.bitcast(w1, jnp.bfloat16)
                        + plsc.bitcast(w2, jnp.bfloat16)
                        + plsc.bitcast(w3, jnp.bfloat16)
                    )
                    ea, eb = plsc.unpack(sb, format=plsc.PackFormat.INTERLEAVED)
                    plsc.store_scatter(out_v, [obase + (2 * p)], ea)
                    plsc.store_scatter(out_v, [obase + (2 * p + 1)], eb)

            pltpu.sync_copy(out_v, out_hbm.at[pl.ds(t0 * _EMBED, _C * _EMBED)])

    return sc_lookup


def kernel(input_ids, factor1_table, factor2_table, frequency_basis, hash_matrix, W, b):
    orig_shape = input_ids.shape
    ids = input_ids.reshape(-1).astype(jnp.int32)
    t1, t2, tf = _make_tables(factor1_table, factor2_table, frequency_basis, W, b)
    t1p = _pack_bf16(t1)
    t2p = _pack_bf16(t2)
    tfp = _pack_bf16(tf)
    out = _build_sc_lookup()(ids, t1p, t2p, tfp, hash_matrix.astype(jnp.int32))
    return out.reshape(orig_shape[0], orig_shape[1], _EMBED)


# speedup vs baseline: 32.6644x; 4.4470x over previous
import functools

import jax
import jax.numpy as jnp
import numpy as np
from jax import lax
from jax.experimental import pallas as pl
from jax.experimental.pallas import tpu as pltpu
from jax.experimental.pallas import tpu_sc as plsc

_VOCAB = 1000000
_EMBED = 64
_NFREQ = 256
_F1 = 1001
_F2 = 1000
_NT = 16384 * 200
_NW = 32
_TPW = _NT // _NW
_C = 512
_NCHUNK = _TPW // _C
_KSUB = _C // 128


def _tables_body(f1_ref, f2_ref, fb_ref, w_ref, b_ref, t1_ref, t2_ref, tf_ref):
    w = w_ref[...]
    dn = (((1,), (1,)), ((), ()))
    hi = jax.lax.Precision.HIGHEST
    t1_ref[...] = lax.dot_general(f1_ref[...], w[:, :32], dn, precision=hi)
    t2_ref[...] = lax.dot_general(f2_ref[...], w[:, 32:], dn, precision=hi)
    tf_ref[...] = 0.1 * lax.dot_general(fb_ref[...], w, dn, precision=hi) + b_ref[...]


@jax.jit
def _make_tables(f1, f2, fb, w, b):
    return pl.pallas_call(
        _tables_body,
        out_shape=(
            jax.ShapeDtypeStruct((_F1, _EMBED), jnp.float32),
            jax.ShapeDtypeStruct((_F2, _EMBED), jnp.float32),
            jax.ShapeDtypeStruct((_NFREQ, _EMBED), jnp.float32),
        ),
    )(f1, f2, fb, w, b.reshape(1, _EMBED))


def _pack_bf16(t):
    bits = lax.bitcast_convert_type(t.astype(jnp.bfloat16), jnp.uint16)
    packed = bits[:, 0::2].astype(jnp.uint32) | (bits[:, 1::2].astype(jnp.uint32) << 16)
    return lax.bitcast_convert_type(packed, jnp.int32).reshape(-1)


@functools.cache
def _build_sc_lookup():
    mesh = plsc.VectorSubcoreMesh(
        core_axis_name="c", subcore_axis_name="s", num_cores=2, num_subcores=16
    )

    @functools.partial(
        pl.kernel,
        out_type=jax.ShapeDtypeStruct((_NT * _EMBED,), jnp.float32),
        mesh=mesh,
        scratch_types=[
            pltpu.VMEM((_F1 * 32,), jnp.int32),
            pltpu.VMEM((_F2 * 32,), jnp.int32),
            pltpu.VMEM((_NFREQ * 32,), jnp.int32),
            pltpu.VMEM((_C,), jnp.int32),
            pltpu.VMEM((_C,), jnp.int32),
            pltpu.VMEM((_C * _EMBED,), jnp.float32),
            pltpu.SemaphoreType.DMA,
        ],
        compiler_params=pltpu.CompilerParams(needs_layout_passes=False),
    )
    def sc_lookup(ids_hbm, t1_hbm, t2_hbm, tf_hbm, hash_hbm, out_hbm,
                  t1_v, t2_v, tf_v, ids_v, h_v, out_v, sem):
        cid = lax.axis_index("c")
        sid = lax.axis_index("s")
        wid = sid * 2 + cid
        pltpu.sync_copy(t1_hbm, t1_v)
        pltpu.sync_copy(t2_hbm, t2_v)
        pltpu.sync_copy(tf_hbm, tf_v)
        tok0 = wid * _TPW

        @pl.loop(0, _NCHUNK)
        def _chunk(ci):
            t0 = tok0 + ci * _C
            pltpu.sync_copy(ids_hbm.at[pl.ds(t0, _C)], ids_v)
            cps = [
                pltpu.async_copy(
                    hash_hbm.at[ids_v.at[pl.ds(j * 128, 128)]],
                    h_v.at[pl.ds(j * 128, 128)],
                    sem,
                )
                for j in range(_KSUB)
            ]
            for cp in cps:
                cp.wait()

            @functools.partial(plsc.parallel_loop, 0, _C // 16, unroll=2)
            def _grp(g):
                v = ids_v[pl.ds(g * 16, 16)]
                h = h_v[pl.ds(g * 16, 16)]
                vf = v.astype(jnp.float32)
                q0 = (vf * np.float32(1.0 / _F1)).astype(jnp.int32)
                r = v - q0 * _F1
                one = jnp.int32(1)
                zero = jnp.int32(0)
                q = q0 + jnp.where(r >= _F1, one, zero) - jnp.where(r < 0, one, zero)
                i1 = v - q * _F1
                i2 = jnp.minimum(q, _F2 - 1)
                b1 = i1 << 5
                b2 = i2 << 5
                b3 = h << 5
                obase = g * (16 * _EMBED) + (lax.iota(jnp.int32, 16) << 6)
                for p in range(32):
                    w1 = plsc.load_gather(t1_v, [b1 + p])
                    w2 = plsc.load_gather(t2_v, [b2 + p])
                    w3 = plsc.load_gather(tf_v, [b3 + p])
                    sb = (
                        plsc.bitcast(w1, jnp.bfloat16)
                        + plsc.bitcast(w2, jnp.bfloat16)
                        + plsc.bitcast(w3, jnp.bfloat16)
                    )
                    ea, eb = plsc.unpack(sb, format=plsc.PackFormat.INTERLEAVED)
                    plsc.store_scatter(out_v, [obase + (2 * p)], ea)
                    plsc.store_scatter(out_v, [obase + (2 * p + 1)], eb)

            pltpu.sync_copy(out_v, out_hbm.at[pl.ds(t0 * _EMBED, _C * _EMBED)])

    return sc_lookup


def kernel(input_ids, factor1_table, factor2_table, frequency_basis, hash_matrix, W, b):
    orig_shape = input_ids.shape
    ids = input_ids.reshape(-1).astype(jnp.int32)
    t1, t2, tf = _make_tables(factor1_table, factor2_table, frequency_basis, W, b)
    t1p = _pack_bf16(t1)
    t2p = _pack_bf16(t2)
    tfp = _pack_bf16(tf)
    out = _build_sc_lookup()(ids, t1p, t2p, tfp, hash_matrix.astype(jnp.int32))
    return out.reshape(orig_shape[0], orig_shape[1], _EMBED)
